# baseline (speedup 1.0000x reference)
import functools

import jax
import jax.numpy as jnp
from jax import lax
from jax.experimental import pallas as pl
from jax.experimental.pallas import tpu as pltpu
from jax.experimental.pallas import tpu_sc as plsc

_B = 32
_H = 12
_L = 577
_NP = _L - 1
_NV = _NP // 16
_DROP = 32
_KEEP_TOTAL = _L - _DROP

_LANES = 16


def _vsort(v):
    return plsc.sort_key_val(v, v)[0]


def _merge_lo(a, b):
    rb = lax.rev(b, (0,))
    return _vsort(jnp.minimum(a, rb))


def _merge_both(a, b):
    rb = lax.rev(b, (0,))
    lo = jnp.minimum(a, rb)
    hi = jnp.maximum(a, rb)
    return _vsort(lo), _vsort(hi)


def _sc_select(attn):
    mesh = plsc.VectorSubcoreMesh(
        core_axis_name="c", subcore_axis_name="s", num_cores=2, num_subcores=16
    )

    @functools.partial(
        pl.kernel,
        out_type=(
            jax.ShapeDtypeStruct((_B, _NP), jnp.float32),
            jax.ShapeDtypeStruct((_B, _NP), jnp.int32),
        ),
        mesh=mesh,
        compiler_params=pltpu.CompilerParams(
            needs_layout_passes=False, use_tc_tiling_on_sc=False
        ),
        scratch_types=[
            pltpu.VMEM((_H, _L), jnp.float32),
            pltpu.VMEM((_NP,), jnp.float32),
            pltpu.VMEM((_NP,), jnp.int32),
            pltpu.SemaphoreType.DMA,
        ],
    )
    def k(attn_hbm, cls_hbm, idx_hbm, rows_v, cls_v, idx_v, sem):
        b = lax.axis_index("s") * 2 + lax.axis_index("c")

        copies = [
            pltpu.async_copy(attn_hbm.at[b, h, 0], rows_v.at[h], sem)
            for h in range(_H)
        ]
        for c in copies:
            c.wait()

        inv_h = jnp.float32(1.0 / _H)
        vals = []
        for i in range(_NV):
            acc = rows_v[0, pl.ds(1 + 16 * i, 16)]
            for h in range(1, _H):
                acc = acc + rows_v[h, pl.ds(1 + 16 * i, 16)]
            v = acc * inv_h
            cls_v[pl.ds(16 * i, 16)] = v
            vals.append(v)

        cls_copy = pltpu.async_copy(cls_v, cls_hbm.at[b], sem)

        s0, s1 = _merge_both(_vsort(vals[0]), _vsort(vals[1]))
        for i in range(2, _NV):
            c_sorted = _vsort(vals[i])
            m_lo = _merge_lo(s1, c_sorted)
            s0, s1 = _merge_both(s0, m_lo)
        t = jnp.max(s1)
        t_vec = jnp.broadcast_to(t, (_LANES,))

        lt_acc = jnp.zeros((_LANES,), jnp.int32)
        eq_acc = jnp.zeros((_LANES,), jnp.int32)
        one = jnp.ones((_LANES,), jnp.int32)
        zero = jnp.zeros((_LANES,), jnp.int32)
        for i in range(_NV):
            v = vals[i]
            lt_acc = lt_acc + jnp.where(v < t_vec, one, zero)
            eq_acc = eq_acc + jnp.where(v == t_vec, one, zero)
        c_lt = jnp.sum(lt_acc)
        m_eq = jnp.sum(eq_acc)
        keep_ties = m_eq - (jnp.int32(_DROP) - c_lt)
        keep_ties_vec = jnp.broadcast_to(keep_ties, (_LANES,))

        idx_v[pl.ds(0, 16)] = zero
        pos = jnp.int32(1)
        q_run = jnp.int32(0)
        base_iota = lax.iota(jnp.int32, _LANES)
        for i in range(_NV):
            v = vals[i]
            eq = v == t_vec
            eq_i32 = jnp.where(eq, one, zero)
            cum = plsc.cumsum(eq_i32) + jnp.broadcast_to(q_run, (_LANES,))
            kept = (v > t_vec) | (eq & (cum <= keep_ties_vec))
            idx = base_iota + jnp.int32(16 * i + 1)
            plsc.store_compressed(idx_v.at[pl.ds(pos, 16)], idx, mask=kept)
            pos = pos + jnp.sum(jnp.where(kept, one, zero))
            q_run = q_run + jnp.sum(eq_i32)

        cls_copy.wait()
        pltpu.sync_copy(idx_v, idx_hbm.at[b])

    return k(attn)


def _copy_body(x_ref, o_ref):
    o_ref[...] = x_ref[...]


def _tc_copy(x):
    return pl.pallas_call(
        _copy_body,
        grid=(_B // 4,),
        in_specs=[pl.BlockSpec((4, _L, 768), lambda i: (i, 0, 0))],
        out_specs=pl.BlockSpec((4, _L, 768), lambda i: (i, 0, 0)),
        out_shape=jax.ShapeDtypeStruct((_B, _L, 768), jnp.float32),
    )(x)


def kernel(x, attn):
    cls_attn, idx_pad = _sc_select(attn)
    full_indices = idx_pad[:, :_KEEP_TOTAL]
    return (_tc_copy(x), full_indices, cls_attn, _L - _DROP)


# speedup vs baseline: 11.1286x; 11.1286x over previous
import functools

import jax
import jax.numpy as jnp
from jax import lax
from jax.experimental import pallas as pl
from jax.experimental.pallas import tpu as pltpu
from jax.experimental.pallas import tpu_sc as plsc

_B = 32
_H = 12
_L = 577
_C = 768
_NP = _L - 1
_NV = _NP // 16
_DROP = 32
_KEEP_TOTAL = _L - _DROP
_PAD = 1024

_LANES = 16


def _vsort(v):
    return plsc.sort_key_val(v, v)[0]


def _merge_lo(a, b):
    rb = lax.rev(b, (0,))
    return _vsort(jnp.minimum(a, rb))


def _merge_both(a, b):
    rb = lax.rev(b, (0,))
    lo = jnp.minimum(a, rb)
    hi = jnp.maximum(a, rb)
    return _vsort(lo), _vsort(hi)


def _tc_body(x_ref, attn_ref, xo_ref, cls_ref, flat_ref):
    xo_ref[...] = x_ref[...]
    m = attn_ref[:, 0, :].sum(axis=0) * jnp.float32(1.0 / _H)
    cls_ref[0, 0, :] = m
    flat_ref[pl.ds(0, _NP)] = m[1:]
    flat_ref[pl.ds(_NP, _PAD - _NP)] = jnp.zeros((_PAD - _NP,), jnp.float32)


def _tc_stage(x, attn):
    attn3 = attn.reshape(_B * _H, _L, _L)
    return pl.pallas_call(
        _tc_body,
        grid=(_B,),
        in_specs=[
            pl.BlockSpec((1, _L, _C), lambda i: (i, 0, 0)),
            pl.BlockSpec((_H, 8, _L), lambda i: (i, 0, 0)),
        ],
        out_specs=[
            pl.BlockSpec((1, _L, _C), lambda i: (i, 0, 0)),
            pl.BlockSpec((1, 1, _L), lambda i: (i, 0, 0)),
            pl.BlockSpec((_PAD,), lambda i: (i,)),
        ],
        out_shape=[
            jax.ShapeDtypeStruct((_B, _L, _C), jnp.float32),
            jax.ShapeDtypeStruct((_B, 1, _L), jnp.float32),
            jax.ShapeDtypeStruct((_B * _PAD,), jnp.float32),
        ],
    )(x, attn3)


def _sc_select(cls_flat):
    mesh = plsc.VectorSubcoreMesh(
        core_axis_name="c", subcore_axis_name="s", num_cores=2, num_subcores=16
    )

    @functools.partial(
        pl.kernel,
        out_type=jax.ShapeDtypeStruct((_B * _PAD,), jnp.int32),
        mesh=mesh,
        compiler_params=pltpu.CompilerParams(
            needs_layout_passes=False, use_tc_tiling_on_sc=False
        ),
        scratch_types=[
            pltpu.VMEM((_PAD,), jnp.float32),
            pltpu.VMEM((_PAD,), jnp.int32),
            pltpu.SemaphoreType.DMA,
        ],
    )
    def k(cls_hbm, idx_hbm, cls_v, idx_v, sem):
        b = lax.axis_index("s") * 2 + lax.axis_index("c")
        pltpu.async_copy(cls_hbm.at[pl.ds(b * _PAD, _PAD)], cls_v, sem).wait()

        vals = [cls_v[pl.ds(16 * i, 16)] for i in range(_NV)]

        s0, s1 = _merge_both(_vsort(vals[0]), _vsort(vals[1]))
        for i in range(2, _NV):
            c_sorted = _vsort(vals[i])
            m_lo = _merge_lo(s1, c_sorted)
            s0, s1 = _merge_both(s0, m_lo)
        t = jnp.max(s1)
        t_vec = jnp.broadcast_to(t, (_LANES,))

        lt_acc = jnp.zeros((_LANES,), jnp.int32)
        eq_acc = jnp.zeros((_LANES,), jnp.int32)
        one = jnp.ones((_LANES,), jnp.int32)
        zero = jnp.zeros((_LANES,), jnp.int32)
        for i in range(_NV):
            v = vals[i]
            lt_acc = lt_acc + jnp.where(v < t_vec, one, zero)
            eq_acc = eq_acc + jnp.where(v == t_vec, one, zero)
        c_lt = jnp.sum(lt_acc)
        m_eq = jnp.sum(eq_acc)
        keep_ties = m_eq - (jnp.int32(_DROP) - c_lt)
        keep_ties_vec = jnp.broadcast_to(keep_ties, (_LANES,))

        idx_v[pl.ds(0, 16)] = zero
        pos = jnp.int32(1)
        q_run = jnp.int32(0)
        base_iota = lax.iota(jnp.int32, _LANES)
        for i in range(_NV):
            v = vals[i]
            eq = v == t_vec
            eq_i32 = jnp.where(eq, one, zero)
            cum = plsc.cumsum(eq_i32) + jnp.broadcast_to(q_run, (_LANES,))
            kept = (v > t_vec) | (eq & (cum <= keep_ties_vec))
            idx = base_iota + jnp.int32(16 * i + 1)
            plsc.store_compressed(idx_v.at[pl.ds(pos, 16)], idx, mask=kept)
            pos = pos + jnp.sum(jnp.where(kept, one, zero))
            q_run = q_run + jnp.sum(eq_i32)

        pltpu.sync_copy(idx_v, idx_hbm.at[pl.ds(b * _PAD, _PAD)])

    return k(cls_flat)


def kernel(x, attn):
    x_out, cls3, cls_flat = _tc_stage(x, attn)
    idx_flat = _sc_select(cls_flat)
    full_indices = idx_flat.reshape(_B, _PAD)[:, :_KEEP_TOTAL]
    cls_attn = cls3[:, 0, 1:]
    return (x_out, full_indices, cls_attn, _L - _DROP)
